# baseline (speedup 1.0000x reference)
import functools

import jax
import jax.numpy as jnp
from jax import lax
from jax.experimental import pallas as pl
from jax.experimental.pallas import tpu as pltpu
from jax.experimental.pallas import tpu_sc as plsc

B, C, H, W = 8, 3, 512, 512
F = 20000
NC, NS, L = 2, 16, 16
NW = NC * NS
ACC = F * C
RPW = H * B // NW
RCH = 8
NCHUNK = RPW // RCH
GPR = W // L


def _sc_scatter_partials(pix_to_face, images):
    mesh = plsc.VectorSubcoreMesh(core_axis_name="c", subcore_axis_name="s")

    @functools.partial(
        pl.kernel,
        out_type=jax.ShapeDtypeStruct((NW, ACC), jnp.float32),
        mesh=mesh,
        compiler_params=pltpu.CompilerParams(needs_layout_passes=False),
        scratch_types=[
            pltpu.VMEM((ACC,), jnp.float32),
            pltpu.VMEM((RCH, W), jnp.int32),
            pltpu.VMEM((RCH, W), jnp.float32),
            pltpu.VMEM((RCH, W), jnp.float32),
            pltpu.VMEM((RCH, W), jnp.float32),
            pltpu.VMEM((RCH, W), jnp.int32),
            pltpu.VMEM((RCH, W), jnp.float32),
            pltpu.VMEM((RCH, W), jnp.float32),
            pltpu.VMEM((RCH, W), jnp.float32),
            pltpu.VMEM((RCH, W), jnp.int32),
            pltpu.VMEM((RCH, W), jnp.float32),
            pltpu.VMEM((RCH, W), jnp.float32),
            pltpu.VMEM((RCH, W), jnp.float32),
            pltpu.SemaphoreType.DMA,
            pltpu.SemaphoreType.DMA,
            pltpu.SemaphoreType.DMA,
        ],
    )
    def sc_kernel(pix_hbm, img_hbm, out_hbm, acc,
                  idxb0, vr0, vg0, vb0, idxb1, vr1, vg1, vb1,
                  idxb2, vr2, vg2, vb2, sem0, sem1, sem2):
        cid = lax.axis_index("c")
        sid = lax.axis_index("s")
        wid = sid * NC + cid
        b = wid // (NW // B)
        q = wid % (NW // B)
        shift = b * (3 * F)
        row0 = q * RPW

        sets = ((idxb0, vr0, vg0, vb0, sem0), (idxb1, vr1, vg1, vb1, sem1),
                (idxb2, vr2, vg2, vb2, sem2))
        NBUF = len(sets)

        def start(k, bufs):
            idxb, vr, vg, vb, sem = bufs
            r0 = row0 + k * RCH
            pltpu.async_copy(pix_hbm.at[b, pl.ds(r0, RCH)], idxb, sem)
            for c, vbuf in ((0, vr), (1, vg), (2, vb)):
                pltpu.async_copy(img_hbm.at[b, c, pl.ds(r0, RCH)], vbuf, sem)

        def drain(bufs):
            idxb, vr, vg, vb, sem = bufs
            pltpu.make_async_copy(pix_hbm.at[b, pl.ds(row0, RCH)], idxb, sem).wait()
            for c, vbuf in ((0, vr), (1, vg), (2, vb)):
                pltpu.make_async_copy(
                    img_hbm.at[b, c, pl.ds(row0, RCH)], vbuf, sem).wait()

        @plsc.parallel_loop(0, ACC // L, unroll=8)
        def _zero(j):
            acc[pl.ds(j * L, L)] = jnp.zeros((L,), jnp.float32)

        for k in range(NBUF - 1):
            start(k, sets[k])
        for k in range(NCHUNK):
            idxb, vr, vg, vb, _ = bufs = sets[k % NBUF]
            if k + NBUF - 1 < NCHUNK:
                start(k + NBUF - 1, sets[(k + NBUF - 1) % NBUF])
            drain(bufs)

            @plsc.parallel_loop(0, RCH * GPR, unroll=8)
            def _group(g):
                row = g >> 5
                sl = pl.ds((g & (GPR - 1)) * L, L)
                i0 = idxb[row, sl] * 3 - shift
                plsc.addupdate_scatter(acc, [i0], vr[row, sl])
                plsc.addupdate_scatter(acc, [i0 + 1], vg[row, sl])
                plsc.addupdate_scatter(acc, [i0 + 2], vb[row, sl])

        pltpu.sync_copy(acc, out_hbm.at[wid])

    return sc_kernel(pix_to_face, images)


def _tc_reduce(partials):
    def body(x_ref, o_ref):
        o_ref[...] = jnp.sum(x_ref[...], axis=0)

    return pl.pallas_call(
        body,
        out_shape=jax.ShapeDtypeStruct((ACC,), jnp.float32),
    )(partials)


def kernel(images, vertices, faces, pix_to_face):
    del vertices, faces
    partials = _sc_scatter_partials(pix_to_face, images)
    colors = _tc_reduce(partials)
    return colors.reshape(F, C)


# speedup vs baseline: 1.0175x; 1.0175x over previous
import functools

import jax
import jax.numpy as jnp
from jax import lax
from jax.experimental import pallas as pl
from jax.experimental.pallas import tpu as pltpu
from jax.experimental.pallas import tpu_sc as plsc

B, C, H, W = 8, 3, 512, 512
F = 20000
NC, NS, L = 2, 16, 16
NW = NC * NS
ACC = F * C
RPW = H * B // NW
RCH = 8
NCHUNK = RPW // RCH
GPR = W // L


def _sc_scatter_partials(pix_to_face, images):
    mesh = plsc.VectorSubcoreMesh(core_axis_name="c", subcore_axis_name="s")

    @functools.partial(
        pl.kernel,
        out_type=jax.ShapeDtypeStruct((NW, ACC), jnp.float32),
        mesh=mesh,
        compiler_params=pltpu.CompilerParams(needs_layout_passes=False),
        scratch_types=[
            pltpu.VMEM((ACC,), jnp.float32),
            pltpu.VMEM((RCH, W), jnp.int32),
            pltpu.VMEM((RCH, W), jnp.float32),
            pltpu.VMEM((RCH, W), jnp.float32),
            pltpu.VMEM((RCH, W), jnp.float32),
            pltpu.VMEM((RCH, W), jnp.int32),
            pltpu.VMEM((RCH, W), jnp.float32),
            pltpu.VMEM((RCH, W), jnp.float32),
            pltpu.VMEM((RCH, W), jnp.float32),
            pltpu.VMEM((RCH, W), jnp.int32),
            pltpu.VMEM((RCH, W), jnp.float32),
            pltpu.VMEM((RCH, W), jnp.float32),
            pltpu.VMEM((RCH, W), jnp.float32),
            pltpu.SemaphoreType.DMA,
            pltpu.SemaphoreType.DMA,
            pltpu.SemaphoreType.DMA,
        ],
    )
    def sc_kernel(pix_hbm, img_hbm, out_hbm, acc,
                  idxb0, vr0, vg0, vb0, idxb1, vr1, vg1, vb1,
                  idxb2, vr2, vg2, vb2, sem0, sem1, sem2):
        cid = lax.axis_index("c")
        sid = lax.axis_index("s")
        wid = sid * NC + cid
        b = wid // (NW // B)
        q = wid % (NW // B)
        shift = b * F
        row0 = q * RPW

        sets = ((idxb0, vr0, vg0, vb0, sem0), (idxb1, vr1, vg1, vb1, sem1),
                (idxb2, vr2, vg2, vb2, sem2))
        NBUF = len(sets)

        def start(k, bufs):
            idxb, vr, vg, vb, sem = bufs
            r0 = row0 + k * RCH
            pltpu.async_copy(pix_hbm.at[b, pl.ds(r0, RCH)], idxb, sem)
            for c, vbuf in ((0, vr), (1, vg), (2, vb)):
                pltpu.async_copy(img_hbm.at[b, c, pl.ds(r0, RCH)], vbuf, sem)

        def drain(bufs):
            idxb, vr, vg, vb, sem = bufs
            pltpu.make_async_copy(pix_hbm.at[b, pl.ds(row0, RCH)], idxb, sem).wait()
            for c, vbuf in ((0, vr), (1, vg), (2, vb)):
                pltpu.make_async_copy(
                    img_hbm.at[b, c, pl.ds(row0, RCH)], vbuf, sem).wait()

        @plsc.parallel_loop(0, ACC // L, unroll=8)
        def _zero(j):
            acc[pl.ds(j * L, L)] = jnp.zeros((L,), jnp.float32)

        for k in range(NBUF - 1):
            start(k, sets[k])
        for k in range(NCHUNK):
            idxb, vr, vg, vb, _ = bufs = sets[k % NBUF]
            if k + NBUF - 1 < NCHUNK:
                start(k + NBUF - 1, sets[(k + NBUF - 1) % NBUF])
            drain(bufs)

            @plsc.parallel_loop(0, RCH * GPR, unroll=8)
            def _group(g):
                row = g >> 5
                sl = pl.ds((g & (GPR - 1)) * L, L)
                i0 = idxb[row, sl] - shift
                plsc.addupdate_scatter(acc, [i0], vr[row, sl])
                plsc.addupdate_scatter(acc, [i0 + F], vg[row, sl])
                plsc.addupdate_scatter(acc, [i0 + 2 * F], vb[row, sl])

        pltpu.sync_copy(acc, out_hbm.at[wid])

    return sc_kernel(pix_to_face, images)


def _tc_reduce(partials):
    def body(x_ref, o_ref):
        cols = [jnp.sum(x_ref[:, c * F:(c + 1) * F], axis=0) for c in range(C)]
        o_ref[...] = jnp.stack(cols, axis=1)

    return pl.pallas_call(
        body,
        out_shape=jax.ShapeDtypeStruct((F, C), jnp.float32),
    )(partials)


def kernel(images, vertices, faces, pix_to_face):
    del vertices, faces
    partials = _sc_scatter_partials(pix_to_face, images)
    return _tc_reduce(partials)


# speedup vs baseline: 1.2966x; 1.2744x over previous
import functools

import jax
import jax.numpy as jnp
from jax import lax
from jax.experimental import pallas as pl
from jax.experimental.pallas import tpu as pltpu
from jax.experimental.pallas import tpu_sc as plsc

B, C, H, W = 8, 3, 512, 512
F = 20000
NC, NS, L = 2, 16, 16
NW = NC * NS
ACC = F * C
RPW = H * B // NW
RCH = 8
NCHUNK = RPW // RCH
GPR = W // L


def _sc_scatter_partials(pix_to_face, images):
    mesh = plsc.VectorSubcoreMesh(core_axis_name="c", subcore_axis_name="s")

    @functools.partial(
        pl.kernel,
        out_type=jax.ShapeDtypeStruct((NW, ACC), jnp.float32),
        mesh=mesh,
        compiler_params=pltpu.CompilerParams(needs_layout_passes=False),
        scratch_types=[
            pltpu.VMEM((ACC,), jnp.float32),
            pltpu.VMEM((RCH, W), jnp.int32),
            pltpu.VMEM((RCH, W), jnp.float32),
            pltpu.VMEM((RCH, W), jnp.float32),
            pltpu.VMEM((RCH, W), jnp.float32),
            pltpu.VMEM((RCH, W), jnp.int32),
            pltpu.VMEM((RCH, W), jnp.float32),
            pltpu.VMEM((RCH, W), jnp.float32),
            pltpu.VMEM((RCH, W), jnp.float32),
            pltpu.VMEM((RCH, W), jnp.int32),
            pltpu.VMEM((RCH, W), jnp.float32),
            pltpu.VMEM((RCH, W), jnp.float32),
            pltpu.VMEM((RCH, W), jnp.float32),
            pltpu.SemaphoreType.DMA,
            pltpu.SemaphoreType.DMA,
            pltpu.SemaphoreType.DMA,
        ],
    )
    def sc_kernel(pix_hbm, img_hbm, out_hbm, acc,
                  idxb0, vr0, vg0, vb0, idxb1, vr1, vg1, vb1,
                  idxb2, vr2, vg2, vb2, sem0, sem1, sem2):
        cid = lax.axis_index("c")
        sid = lax.axis_index("s")
        wid = sid * NC + cid
        b = wid // (NW // B)
        q = wid % (NW // B)
        shift = b * F
        row0 = q * RPW

        sets = ((idxb0, vr0, vg0, vb0, sem0), (idxb1, vr1, vg1, vb1, sem1),
                (idxb2, vr2, vg2, vb2, sem2))
        NBUF = len(sets)

        def start(k, bufs):
            idxb, vr, vg, vb, sem = bufs
            r0 = row0 + k * RCH
            pltpu.async_copy(pix_hbm.at[b, pl.ds(r0, RCH)], idxb, sem)
            for c, vbuf in ((0, vr), (1, vg), (2, vb)):
                pltpu.async_copy(img_hbm.at[b, c, pl.ds(r0, RCH)], vbuf, sem)

        def drain(bufs):
            idxb, vr, vg, vb, sem = bufs
            pltpu.make_async_copy(pix_hbm.at[b, pl.ds(row0, RCH)], idxb, sem).wait()
            for c, vbuf in ((0, vr), (1, vg), (2, vb)):
                pltpu.make_async_copy(
                    img_hbm.at[b, c, pl.ds(row0, RCH)], vbuf, sem).wait()

        @plsc.parallel_loop(0, ACC // L, unroll=8)
        def _zero(j):
            acc[pl.ds(j * L, L)] = jnp.zeros((L,), jnp.float32)

        for k in range(NBUF - 1):
            start(k, sets[k])
        for k in range(NCHUNK):
            idxb, vr, vg, vb, _ = bufs = sets[k % NBUF]
            if k + NBUF - 1 < NCHUNK:
                start(k + NBUF - 1, sets[(k + NBUF - 1) % NBUF])
            drain(bufs)

            @plsc.parallel_loop(0, RCH * GPR, unroll=8)
            def _group(g):
                row = g >> 5
                sl = pl.ds((g & (GPR - 1)) * L, L)
                i0 = idxb[row, sl] - shift
                plsc.addupdate_scatter(acc, [i0], vr[row, sl])
                plsc.addupdate_scatter(acc, [i0 + F], vg[row, sl])
                plsc.addupdate_scatter(acc, [i0 + 2 * F], vb[row, sl])

        pltpu.sync_copy(acc, out_hbm.at[wid])

    return sc_kernel(pix_to_face, images)


def _tc_reduce(partials):
    def body(x_ref, o_ref):
        for c in range(C):
            o_ref[c, :] = jnp.sum(x_ref[:, c * F:(c + 1) * F], axis=0)

    return pl.pallas_call(
        body,
        out_shape=jax.ShapeDtypeStruct((C, F), jnp.float32),
    )(partials)


def kernel(images, vertices, faces, pix_to_face):
    del vertices, faces
    partials = _sc_scatter_partials(pix_to_face, images)
    return _tc_reduce(partials).T


# speedup vs baseline: 1.4243x; 1.0985x over previous
import functools

import jax
import jax.numpy as jnp
from jax import lax
from jax.experimental import pallas as pl
from jax.experimental.pallas import tpu as pltpu
from jax.experimental.pallas import tpu_sc as plsc

B, C, H, W = 8, 3, 512, 512
F = 20000
NC, NS, L = 2, 16, 16
NW = NC * NS
ACC = F * C
RPW = H * B // NW
RCH = 8
NCHUNK = RPW // RCH
GPR = W // L


def _sc_scatter_partials(pix_to_face, images):
    mesh = plsc.VectorSubcoreMesh(core_axis_name="c", subcore_axis_name="s")

    @functools.partial(
        pl.kernel,
        out_type=jax.ShapeDtypeStruct((NW, ACC), jnp.float32),
        mesh=mesh,
        compiler_params=pltpu.CompilerParams(needs_layout_passes=False),
        scratch_types=[
            pltpu.VMEM((ACC,), jnp.float32),
            pltpu.VMEM((RCH, W), jnp.int32),
            pltpu.VMEM((RCH, W), jnp.float32),
            pltpu.VMEM((RCH, W), jnp.float32),
            pltpu.VMEM((RCH, W), jnp.float32),
            pltpu.VMEM((RCH, W), jnp.int32),
            pltpu.VMEM((RCH, W), jnp.float32),
            pltpu.VMEM((RCH, W), jnp.float32),
            pltpu.VMEM((RCH, W), jnp.float32),
            pltpu.SemaphoreType.DMA,
            pltpu.SemaphoreType.DMA,
        ],
    )
    def sc_kernel(pix_hbm, img_hbm, out_hbm, acc,
                  idxb0, vr0, vg0, vb0, idxb1, vr1, vg1, vb1, sem0, sem1):
        cid = lax.axis_index("c")
        sid = lax.axis_index("s")
        wid = sid * NC + cid
        b = wid // (NW // B)
        q = wid % (NW // B)
        shift = b * F
        row0 = q * RPW

        sets = ((idxb0, vr0, vg0, vb0, sem0), (idxb1, vr1, vg1, vb1, sem1))

        def start(k, bufs):
            idxb, vr, vg, vb, sem = bufs
            r0 = row0 + k * RCH
            pltpu.async_copy(pix_hbm.at[b, pl.ds(r0, RCH)], idxb, sem)
            for c, vbuf in ((0, vr), (1, vg), (2, vb)):
                pltpu.async_copy(img_hbm.at[b, c, pl.ds(r0, RCH)], vbuf, sem)

        def drain(bufs):
            idxb, vr, vg, vb, sem = bufs
            pltpu.make_async_copy(pix_hbm.at[b, pl.ds(row0, RCH)], idxb, sem).wait()
            for c, vbuf in ((0, vr), (1, vg), (2, vb)):
                pltpu.make_async_copy(
                    img_hbm.at[b, c, pl.ds(row0, RCH)], vbuf, sem).wait()

        def compute(bufs):
            idxb, vr, vg, vb, _ = bufs

            @plsc.parallel_loop(0, RCH * GPR, unroll=16)
            def _group(g):
                row = g >> 5
                sl = pl.ds((g & (GPR - 1)) * L, L)
                i0 = idxb[row, sl] - shift
                plsc.addupdate_scatter(acc, [i0], vr[row, sl])
                plsc.addupdate_scatter(acc, [i0 + F], vg[row, sl])
                plsc.addupdate_scatter(acc, [i0 + 2 * F], vb[row, sl])

        start(0, sets[0])
        start(1, sets[1])

        @plsc.parallel_loop(0, ACC // L, unroll=8)
        def _zero(j):
            acc[pl.ds(j * L, L)] = jnp.zeros((L,), jnp.float32)

        def pair_body(kk, carry):
            k = kk * 2
            for j in range(2):
                bufs = sets[j]
                drain(bufs)
                compute(bufs)
                start(k + j + 2, bufs)
            return carry

        lax.fori_loop(0, NCHUNK // 2 - 1, pair_body, 0)
        for j in range(2):
            drain(sets[j])
            compute(sets[j])

        pltpu.sync_copy(acc, out_hbm.at[wid])

    return sc_kernel(pix_to_face, images)


def _tc_reduce(partials):
    def body(x_ref, o_ref):
        for c in range(C):
            o_ref[c, :] = jnp.sum(x_ref[:, c * F:(c + 1) * F], axis=0)

    return pl.pallas_call(
        body,
        out_shape=jax.ShapeDtypeStruct((C, F), jnp.float32),
    )(partials)


def kernel(images, vertices, faces, pix_to_face):
    del vertices, faces
    partials = _sc_scatter_partials(pix_to_face, images)
    return _tc_reduce(partials).T


# speedup vs baseline: 1.4251x; 1.0005x over previous
import functools

import jax
import jax.numpy as jnp
from jax import lax
from jax.experimental import pallas as pl
from jax.experimental.pallas import tpu as pltpu
from jax.experimental.pallas import tpu_sc as plsc

B, C, H, W = 8, 3, 512, 512
F = 20000
NC, NS, L = 2, 16, 16
NW = NC * NS
ACC = F * C
RPW = H * B // NW
RCH = 8
NCHUNK = RPW // RCH
GPR = W // L


def _sc_scatter_partials(pix_to_face, images):
    mesh = plsc.VectorSubcoreMesh(core_axis_name="c", subcore_axis_name="s")

    @functools.partial(
        pl.kernel,
        out_type=jax.ShapeDtypeStruct((NW, ACC), jnp.float32),
        mesh=mesh,
        compiler_params=pltpu.CompilerParams(needs_layout_passes=False),
        scratch_types=[
            pltpu.VMEM((ACC,), jnp.float32),
            pltpu.VMEM((RCH, W), jnp.int32),
            pltpu.VMEM((RCH, W), jnp.float32),
            pltpu.VMEM((RCH, W), jnp.float32),
            pltpu.VMEM((RCH, W), jnp.float32),
            pltpu.VMEM((RCH, W), jnp.int32),
            pltpu.VMEM((RCH, W), jnp.float32),
            pltpu.VMEM((RCH, W), jnp.float32),
            pltpu.VMEM((RCH, W), jnp.float32),
            pltpu.SemaphoreType.DMA,
            pltpu.SemaphoreType.DMA,
        ],
    )
    def sc_kernel(pix_hbm, img_hbm, out_hbm, acc,
                  idxb0, vr0, vg0, vb0, idxb1, vr1, vg1, vb1, sem0, sem1):
        cid = lax.axis_index("c")
        sid = lax.axis_index("s")
        wid = sid * NC + cid
        b = wid // (NW // B)
        q = wid % (NW // B)
        shift = b * F
        row0 = q * RPW

        sets = ((idxb0, vr0, vg0, vb0, sem0), (idxb1, vr1, vg1, vb1, sem1))

        def start(k, bufs):
            idxb, vr, vg, vb, sem = bufs
            r0 = row0 + k * RCH
            pltpu.async_copy(pix_hbm.at[b, pl.ds(r0, RCH)], idxb, sem)
            for c, vbuf in ((0, vr), (1, vg), (2, vb)):
                pltpu.async_copy(img_hbm.at[b, c, pl.ds(r0, RCH)], vbuf, sem)

        def drain(bufs):
            idxb, vr, vg, vb, sem = bufs
            pltpu.make_async_copy(pix_hbm.at[b, pl.ds(row0, RCH)], idxb, sem).wait()
            for c, vbuf in ((0, vr), (1, vg), (2, vb)):
                pltpu.make_async_copy(
                    img_hbm.at[b, c, pl.ds(row0, RCH)], vbuf, sem).wait()

        def compute(bufs):
            idxb, vr, vg, vb, _ = bufs

            @plsc.parallel_loop(0, RCH * GPR, unroll=16)
            def _group(g):
                row = g >> 5
                sl = pl.ds((g & (GPR - 1)) * L, L)
                i0 = idxb[row, sl] - shift
                plsc.addupdate_scatter(acc, [i0], vr[row, sl])
                plsc.addupdate_scatter(acc, [i0 + F], vg[row, sl])
                plsc.addupdate_scatter(acc, [i0 + 2 * F], vb[row, sl])

        start(0, sets[0])
        start(1, sets[1])

        @plsc.parallel_loop(0, ACC // L, unroll=10)
        def _zero(j):
            acc[pl.ds(j * L, L)] = jnp.zeros((L,), jnp.float32)

        def pair_body(kk, carry):
            k = kk * 2
            for j in range(2):
                bufs = sets[j]
                drain(bufs)
                compute(bufs)
                start(k + j + 2, bufs)
            return carry

        lax.fori_loop(0, NCHUNK // 2 - 1, pair_body, 0)
        for j in range(2):
            drain(sets[j])
            compute(sets[j])

        pltpu.sync_copy(acc, out_hbm.at[wid])

    return sc_kernel(pix_to_face, images)


def _tc_reduce(partials):
    def body(x_ref, o_ref):
        for c in range(C):
            o_ref[c, :] = jnp.sum(x_ref[:, c * F:(c + 1) * F], axis=0)

    return pl.pallas_call(
        body,
        out_shape=jax.ShapeDtypeStruct((C, F), jnp.float32),
    )(partials)


def kernel(images, vertices, faces, pix_to_face):
    del vertices, faces
    partials = _sc_scatter_partials(pix_to_face, images)
    return _tc_reduce(partials).T
